# baseline (speedup 1.0000x reference)
import jax
import jax.numpy as jnp
from jax import lax
from jax.experimental import pallas as pl
from jax.experimental.pallas import tpu as pltpu
from jax.experimental.pallas import tpu_sc as plsc

N_NODES = 50000
D = 128
E = 300000
N_CLS = 8

NTILES = 16
NCORES = 2
NCHUNK = 8
DC = D // NCHUNK
KE = 2048
KC = 512
NBATCH = 5
EPT = NBATCH * KE
E_PAD = EPT * NTILES * NCORES
ZR = 784
NT = 50176
TB = NT + 8
DUMP = NT
STRIPE = NT // NTILES
BLK = 392


def _seg_sums(tabs, src2d, dst2d):
    def body(*refs):
        tabs_in = refs[:NCHUNK]
        (src_hbm, dst_hbm, sums_o, cnt_o,
         src_idx, dst_idx, cnt_idx, rows, ones, zbuf, acc, sem) = refs[NCHUNK:]
        cid = lax.axis_index("c")
        sid = lax.axis_index("s")
        e0 = (cid * NTILES + sid) * EPT

        zf = jnp.zeros((16,), jnp.float32)
        of = jnp.ones((16,), jnp.float32)

        def init_row(r, _):
            zbuf[r, pl.ds(0, 16)] = zf
            return 0

        lax.fori_loop(0, ZR, init_row, 0)

        def init_ones(r, _):
            ones[r, pl.ds(0, 16)] = of
            return 0

        lax.fori_loop(0, KC, init_ones, 0)

        def zero_acc():
            z0 = sid * STRIPE
            for j in range(STRIPE // ZR):
                pltpu.sync_copy(zbuf, acc.at[pl.ds(z0 + j * ZR, ZR)])
            plsc.subcore_barrier()

        def writeout(dst_view):
            plsc.subcore_barrier()
            w0 = sid * STRIPE
            pltpu.sync_copy(acc.at[pl.ds(w0, STRIPE)],
                            dst_view.at[pl.ds(w0, STRIPE)])
            plsc.subcore_barrier()

        def zero_dump():
            pltpu.sync_copy(zbuf.at[pl.ds(0, 8)], acc.at[pl.ds(TB - 8, 8)])

        for c in range(NCHUNK):
            tab = tabs_in[c]
            zero_acc()
            zero_dump()

            def biter(b, _):
                pltpu.sync_copy(src_hbm.at[pl.ds(e0 + b * KE, KE)], src_idx)
                pltpu.sync_copy(dst_hbm.at[pl.ds(e0 + b * KE, KE)], dst_idx)
                pltpu.async_copy(tab.at[src_idx], rows, sem).wait()
                pltpu.sync_copy(rows, acc.at[dst_idx], add=True)
                return 0

            lax.fori_loop(0, NBATCH, biter, 0)
            writeout(sums_o.at[c, cid])

        zero_acc()
        zero_dump()

        def citer(b, _):
            pltpu.sync_copy(dst_hbm.at[pl.ds(e0 + b * KC, KC)], cnt_idx)
            pltpu.sync_copy(ones, acc.at[cnt_idx], add=True)
            return 0

        lax.fori_loop(0, EPT // KC, citer, 0)
        writeout(cnt_o.at[cid])

    k = pl.kernel(
        body,
        out_type=(
            jax.ShapeDtypeStruct((NCHUNK, NCORES, NT, DC), jnp.float32),
            jax.ShapeDtypeStruct((NCORES, NT, DC), jnp.float32),
        ),
        mesh=plsc.VectorSubcoreMesh(core_axis_name="c", subcore_axis_name="s"),
        scratch_types=(
            pltpu.VMEM((KE,), jnp.int32),
            pltpu.VMEM((KE,), jnp.int32),
            pltpu.VMEM((KC,), jnp.int32),
            pltpu.VMEM((KE, DC), jnp.float32),
            pltpu.VMEM((KC, DC), jnp.float32),
            pltpu.VMEM((ZR, DC), jnp.float32),
            pltpu.VMEM_SHARED((TB, DC), jnp.float32),
            pltpu.SemaphoreType.DMA,
        ),
        compiler_params=pltpu.CompilerParams(use_tc_tiling_on_sc=False),
    )
    return k(*tabs, src2d, dst2d)


def _stage_in(x, w, b):
    n = x.shape[0]
    blk = 400 if n % 400 == 0 else BLK

    def kern(x_ref, w_ref, b_ref, *outs):
        h = (jnp.dot(x_ref[...], w_ref[...],
                     preferred_element_type=jnp.float32) + b_ref[...])
        for c in range(NCHUNK):
            outs[c][...] = h[:, c * DC:(c + 1) * DC]

    return pl.pallas_call(
        kern,
        grid=(n // blk,),
        in_specs=[
            pl.BlockSpec((blk, D), lambda i: (i, 0)),
            pl.BlockSpec((D, D), lambda i: (0, 0)),
            pl.BlockSpec((1, D), lambda i: (0, 0)),
        ],
        out_specs=[pl.BlockSpec((blk, DC), lambda i: (i, 0))
                   for _ in range(NCHUNK)],
        out_shape=[jax.ShapeDtypeStruct((n, DC), jnp.float32)
                   for _ in range(NCHUNK)],
    )(x, w, b.reshape(1, D))


def _stage_merge(sums, cnt, w, b, relu, split_out):
    d_out = w.shape[1]

    def kern(s_ref, c_ref, w_ref, b_ref, *outs):
        hs = [s_ref[c, 0] + s_ref[c, 1] for c in range(NCHUNK)]
        h = jnp.concatenate(hs, axis=1)
        cnt_b = c_ref[0] + c_ref[1]
        h = h / jnp.maximum(cnt_b[:, 0:1], 1.0)
        if relu:
            h = jnp.where(h >= 0, h, 0.01 * h)
        o = (jnp.dot(h, w_ref[...], preferred_element_type=jnp.float32)
             + b_ref[...])
        if split_out:
            for c in range(NCHUNK):
                outs[c][...] = o[:, c * DC:(c + 1) * DC]
        else:
            outs[0][...] = o

    if split_out:
        out_specs = [pl.BlockSpec((BLK, DC), lambda i: (i, 0))
                     for _ in range(NCHUNK)]
        out_shape = [jax.ShapeDtypeStruct((NT, DC), jnp.float32)
                     for _ in range(NCHUNK)]
    else:
        out_specs = [pl.BlockSpec((BLK, d_out), lambda i: (i, 0))]
        out_shape = [jax.ShapeDtypeStruct((NT, d_out), jnp.float32)]

    return pl.pallas_call(
        kern,
        grid=(NT // BLK,),
        in_specs=[
            pl.BlockSpec((NCHUNK, NCORES, BLK, DC), lambda i: (0, 0, i, 0)),
            pl.BlockSpec((NCORES, BLK, DC), lambda i: (0, i, 0)),
            pl.BlockSpec((D, d_out), lambda i: (0, 0)),
            pl.BlockSpec((1, d_out), lambda i: (0, 0)),
        ],
        out_specs=out_specs,
        out_shape=out_shape,
    )(sums, cnt, w, b.reshape(1, d_out))


def _edges_pad(ei):
    npad = E_PAD - E
    src = jnp.concatenate([ei[0].astype(jnp.int32),
                           jnp.zeros((npad,), jnp.int32)])
    dst = jnp.concatenate([ei[1].astype(jnp.int32),
                           jnp.full((npad,), DUMP, jnp.int32)])
    return src, dst


def kernel(features, edge_index_u2t, edge_index_t2u, embed_user,
           W0_u2t, b0_u2t, W0_t2u, b0_t2u,
           W1_u2t, b1_u2t, W1_t2u, b1_t2u, Wc, bc):
    src_t2u, dst_t2u = _edges_pad(edge_index_t2u)
    src_u2t, dst_u2t = _edges_pad(edge_index_u2t)

    wh0 = _stage_in(features, W0_t2u, b0_t2u)
    sums_u, cnt_u = _seg_sums(wh0, src_t2u, dst_t2u)
    wh1 = _stage_merge(sums_u, cnt_u, W1_u2t, b1_u2t, relu=True,
                       split_out=True)
    sums_t, cnt_t = _seg_sums(wh1, src_u2t, dst_u2t)
    out = _stage_merge(sums_t, cnt_t, Wc, bc, relu=False, split_out=False)[0]
    return out[:N_NODES]


# speedup vs baseline: 1.0096x; 1.0096x over previous
import jax
import jax.numpy as jnp
from jax import lax
from jax.experimental import pallas as pl
from jax.experimental.pallas import tpu as pltpu
from jax.experimental.pallas import tpu_sc as plsc

N_NODES = 50000
D = 128
E = 300000
N_CLS = 8

NTILES = 16
NCORES = 2
NCHUNK = 8
DC = D // NCHUNK
KE = 1024
KC = 512
NBATCH = 10
EPT = NBATCH * KE
E_PAD = EPT * NTILES * NCORES
ZR = 784
NT = 50176
TB = NT + 8
DUMP = NT
STRIPE = NT // NTILES
BLK = 392


def _seg_sums(tabs, src2d, dst2d):
    def body(*refs):
        tabs_in = refs[:NCHUNK]
        (src_hbm, dst_hbm, sums_o, cnt_o,
         src_idx0, src_idx1, dst_idx0, dst_idx1, cnt_idx,
         rows0, rows1, ones, zbuf, acc, sem0, sem1) = refs[NCHUNK:]
        src_idx = (src_idx0, src_idx1)
        dst_idx = (dst_idx0, dst_idx1)
        rows = (rows0, rows1)
        sems = (sem0, sem1)
        cid = lax.axis_index("c")
        sid = lax.axis_index("s")
        e0 = (cid * NTILES + sid) * EPT

        zf = jnp.zeros((16,), jnp.float32)
        of = jnp.ones((16,), jnp.float32)

        def init_row(r, _):
            zbuf[r, pl.ds(0, 16)] = zf
            return 0

        lax.fori_loop(0, ZR, init_row, 0)

        def init_ones(r, _):
            ones[r, pl.ds(0, 16)] = of
            return 0

        lax.fori_loop(0, KC, init_ones, 0)

        def zero_acc():
            z0 = sid * STRIPE
            for j in range(STRIPE // ZR):
                pltpu.sync_copy(zbuf, acc.at[pl.ds(z0 + j * ZR, ZR)])
            plsc.subcore_barrier()

        def writeout(dst_view):
            plsc.subcore_barrier()
            w0 = sid * STRIPE
            pltpu.sync_copy(acc.at[pl.ds(w0, STRIPE)],
                            dst_view.at[pl.ds(w0, STRIPE)])
            plsc.subcore_barrier()

        def zero_dump():
            pltpu.sync_copy(zbuf.at[pl.ds(0, 8)], acc.at[pl.ds(TB - 8, 8)])

        for c in range(NCHUNK):
            tab = tabs_in[c]
            zero_acc()
            zero_dump()

            pltpu.sync_copy(src_hbm.at[pl.ds(e0, KE)], src_idx[0])
            pltpu.sync_copy(dst_hbm.at[pl.ds(e0, KE)], dst_idx[0])
            g = pltpu.async_copy(tab.at[src_idx[0]], rows[0], sems[0])
            for b in range(NBATCH):
                p = b % 2
                q = (b + 1) % 2
                if b + 1 < NBATCH:
                    pltpu.sync_copy(
                        src_hbm.at[pl.ds(e0 + (b + 1) * KE, KE)], src_idx[q])
                    pltpu.sync_copy(
                        dst_hbm.at[pl.ds(e0 + (b + 1) * KE, KE)], dst_idx[q])
                    g.wait()
                    g = pltpu.async_copy(tab.at[src_idx[q]], rows[q], sems[q])
                else:
                    g.wait()
                pltpu.sync_copy(rows[p], acc.at[dst_idx[p]], add=True)
            writeout(sums_o.at[c, cid])

        zero_acc()
        zero_dump()

        def citer(b, _):
            pltpu.sync_copy(dst_hbm.at[pl.ds(e0 + b * KC, KC)], cnt_idx)
            pltpu.sync_copy(ones, acc.at[cnt_idx], add=True)
            return 0

        lax.fori_loop(0, EPT // KC, citer, 0)
        writeout(cnt_o.at[cid])

    k = pl.kernel(
        body,
        out_type=(
            jax.ShapeDtypeStruct((NCHUNK, NCORES, NT, DC), jnp.float32),
            jax.ShapeDtypeStruct((NCORES, NT, DC), jnp.float32),
        ),
        mesh=plsc.VectorSubcoreMesh(core_axis_name="c", subcore_axis_name="s"),
        scratch_types=(
            pltpu.VMEM((KE,), jnp.int32),
            pltpu.VMEM((KE,), jnp.int32),
            pltpu.VMEM((KE,), jnp.int32),
            pltpu.VMEM((KE,), jnp.int32),
            pltpu.VMEM((KC,), jnp.int32),
            pltpu.VMEM((KE, DC), jnp.float32),
            pltpu.VMEM((KE, DC), jnp.float32),
            pltpu.VMEM((KC, DC), jnp.float32),
            pltpu.VMEM((ZR, DC), jnp.float32),
            pltpu.VMEM_SHARED((TB, DC), jnp.float32),
            pltpu.SemaphoreType.DMA,
            pltpu.SemaphoreType.DMA,
        ),
        compiler_params=pltpu.CompilerParams(use_tc_tiling_on_sc=False),
    )
    return k(*tabs, src2d, dst2d)


def _stage_in(x, w, b):
    n = x.shape[0]
    blk = 400 if n % 400 == 0 else BLK

    def kern(x_ref, w_ref, b_ref, *outs):
        h = (jnp.dot(x_ref[...], w_ref[...],
                     preferred_element_type=jnp.float32) + b_ref[...])
        for c in range(NCHUNK):
            outs[c][...] = h[:, c * DC:(c + 1) * DC]

    return pl.pallas_call(
        kern,
        grid=(n // blk,),
        in_specs=[
            pl.BlockSpec((blk, D), lambda i: (i, 0)),
            pl.BlockSpec((D, D), lambda i: (0, 0)),
            pl.BlockSpec((1, D), lambda i: (0, 0)),
        ],
        out_specs=[pl.BlockSpec((blk, DC), lambda i: (i, 0))
                   for _ in range(NCHUNK)],
        out_shape=[jax.ShapeDtypeStruct((n, DC), jnp.float32)
                   for _ in range(NCHUNK)],
    )(x, w, b.reshape(1, D))


def _stage_merge(sums, cnt, w, b, relu, split_out):
    d_out = w.shape[1]

    def kern(s_ref, c_ref, w_ref, b_ref, *outs):
        hs = [s_ref[c, 0] + s_ref[c, 1] for c in range(NCHUNK)]
        h = jnp.concatenate(hs, axis=1)
        cnt_b = c_ref[0] + c_ref[1]
        h = h / jnp.maximum(cnt_b[:, 0:1], 1.0)
        if relu:
            h = jnp.where(h >= 0, h, 0.01 * h)
        o = (jnp.dot(h, w_ref[...], preferred_element_type=jnp.float32)
             + b_ref[...])
        if split_out:
            for c in range(NCHUNK):
                outs[c][...] = o[:, c * DC:(c + 1) * DC]
        else:
            outs[0][...] = o

    if split_out:
        out_specs = [pl.BlockSpec((BLK, DC), lambda i: (i, 0))
                     for _ in range(NCHUNK)]
        out_shape = [jax.ShapeDtypeStruct((NT, DC), jnp.float32)
                     for _ in range(NCHUNK)]
    else:
        out_specs = [pl.BlockSpec((BLK, d_out), lambda i: (i, 0))]
        out_shape = [jax.ShapeDtypeStruct((NT, d_out), jnp.float32)]

    return pl.pallas_call(
        kern,
        grid=(NT // BLK,),
        in_specs=[
            pl.BlockSpec((NCHUNK, NCORES, BLK, DC), lambda i: (0, 0, i, 0)),
            pl.BlockSpec((NCORES, BLK, DC), lambda i: (0, i, 0)),
            pl.BlockSpec((D, d_out), lambda i: (0, 0)),
            pl.BlockSpec((1, d_out), lambda i: (0, 0)),
        ],
        out_specs=out_specs,
        out_shape=out_shape,
    )(sums, cnt, w, b.reshape(1, d_out))


def _edges_pad(ei):
    npad = E_PAD - E
    src = jnp.concatenate([ei[0].astype(jnp.int32),
                           jnp.zeros((npad,), jnp.int32)])
    dst = jnp.concatenate([ei[1].astype(jnp.int32),
                           jnp.full((npad,), DUMP, jnp.int32)])
    return src, dst


def kernel(features, edge_index_u2t, edge_index_t2u, embed_user,
           W0_u2t, b0_u2t, W0_t2u, b0_t2u,
           W1_u2t, b1_u2t, W1_t2u, b1_t2u, Wc, bc):
    src_t2u, dst_t2u = _edges_pad(edge_index_t2u)
    src_u2t, dst_u2t = _edges_pad(edge_index_u2t)

    wh0 = _stage_in(features, W0_t2u, b0_t2u)
    sums_u, cnt_u = _seg_sums(wh0, src_t2u, dst_t2u)
    wh1 = _stage_merge(sums_u, cnt_u, W1_u2t, b1_u2t, relu=True,
                       split_out=True)
    sums_t, cnt_t = _seg_sums(wh1, src_u2t, dst_u2t)
    out = _stage_merge(sums_t, cnt_t, Wc, bc, relu=False, split_out=False)[0]
    return out[:N_NODES]


# speedup vs baseline: 1.7326x; 1.7161x over previous
import jax
import jax.numpy as jnp
from jax import lax
from jax.experimental import pallas as pl
from jax.experimental.pallas import tpu as pltpu
from jax.experimental.pallas import tpu_sc as plsc

N_NODES = 50000
D = 128
E = 300000
N_CLS = 8

NTILES = 16
NCORES = 2
NCHUNK = 8
DC = D // NCHUNK
KE = 512
KC = 256
NBATCH = 20
EPT = NBATCH * KE
E_PAD = EPT * NTILES * NCORES
ZR = 392
TSTRIPE = 3136
NT = 50176
TB = NT + 8
DUMP = NT
STRIPE = NT // NTILES
BLK = 392


def _seg_sums(tabs, src2d, dst2d):
    def body(*refs):
        tabs_in = refs[:NCHUNK]
        (src_hbm, dst_hbm, sums_o, cnt_o,
         src_idx0, src_idx1, dst_idx0, dst_idx1, cnt_idx,
         rows0, rows1, ones, zbuf, tab_sp, acc, sem0, sem1) = refs[NCHUNK:]
        src_idx = (src_idx0, src_idx1)
        dst_idx = (dst_idx0, dst_idx1)
        rows = (rows0, rows1)
        sems = (sem0, sem1)
        cid = lax.axis_index("c")
        sid = lax.axis_index("s")
        e0 = (cid * NTILES + sid) * EPT

        zf = jnp.zeros((16,), jnp.float32)
        of = jnp.ones((16,), jnp.float32)

        def init_row(r, _):
            zbuf[r, pl.ds(0, 16)] = zf
            return 0

        lax.fori_loop(0, ZR, init_row, 0)

        def init_ones(r, _):
            ones[r, pl.ds(0, 16)] = of
            return 0

        lax.fori_loop(0, KC, init_ones, 0)

        def zero_acc():
            z0 = sid * STRIPE
            for j in range(STRIPE // ZR):
                pltpu.sync_copy(zbuf, acc.at[pl.ds(z0 + j * ZR, ZR)])
            plsc.subcore_barrier()

        def writeout(dst_view):
            plsc.subcore_barrier()
            w0 = sid * STRIPE
            pltpu.sync_copy(acc.at[pl.ds(w0, STRIPE)],
                            dst_view.at[pl.ds(w0, STRIPE)])
            plsc.subcore_barrier()

        def zero_dump():
            pltpu.sync_copy(zbuf.at[pl.ds(0, 8)], acc.at[pl.ds(TB - 8, 8)])

        for c in range(NCHUNK):
            tab = tabs_in[c]
            t0 = jnp.minimum(sid * TSTRIPE, N_NODES - TSTRIPE)
            pltpu.sync_copy(tab.at[pl.ds(t0, TSTRIPE)],
                            tab_sp.at[pl.ds(t0, TSTRIPE)])
            zero_acc()
            zero_dump()
            plsc.subcore_barrier()

            def biter2(i, _):
                for u in range(2):
                    b = i * 2 + u
                    p = u
                    q = 1 - u
                    pltpu.sync_copy(
                        src_hbm.at[pl.ds(e0 + b * KE, KE)], src_idx[p])
                    pltpu.sync_copy(
                        dst_hbm.at[pl.ds(e0 + b * KE, KE)], dst_idx[p])
                    g = pltpu.async_copy(tab_sp.at[src_idx[p]], rows[p],
                                         sems[p])
                    g.wait()
                    pltpu.sync_copy(rows[p], acc.at[dst_idx[p]], add=True)
                return 0

            lax.fori_loop(0, NBATCH // 2, biter2, 0)
            writeout(sums_o.at[c, cid])

        zero_acc()
        zero_dump()

        def citer(b, _):
            pltpu.sync_copy(dst_hbm.at[pl.ds(e0 + b * KC, KC)], cnt_idx)
            pltpu.sync_copy(ones, acc.at[cnt_idx], add=True)
            return 0

        lax.fori_loop(0, EPT // KC, citer, 0)
        writeout(cnt_o.at[cid])

    k = pl.kernel(
        body,
        out_type=(
            jax.ShapeDtypeStruct((NCHUNK, NCORES, NT, DC), jnp.float32),
            jax.ShapeDtypeStruct((NCORES, NT, DC), jnp.float32),
        ),
        mesh=plsc.VectorSubcoreMesh(core_axis_name="c", subcore_axis_name="s"),
        scratch_types=(
            pltpu.VMEM((KE,), jnp.int32),
            pltpu.VMEM((KE,), jnp.int32),
            pltpu.VMEM((KE,), jnp.int32),
            pltpu.VMEM((KE,), jnp.int32),
            pltpu.VMEM((KC,), jnp.int32),
            pltpu.VMEM((KE, DC), jnp.float32),
            pltpu.VMEM((KE, DC), jnp.float32),
            pltpu.VMEM((KC, DC), jnp.float32),
            pltpu.VMEM((ZR, DC), jnp.float32),
            pltpu.VMEM_SHARED((N_NODES, DC), jnp.float32),
            pltpu.VMEM_SHARED((TB, DC), jnp.float32),
            pltpu.SemaphoreType.DMA,
            pltpu.SemaphoreType.DMA,
        ),
        compiler_params=pltpu.CompilerParams(use_tc_tiling_on_sc=False),
    )
    return k(*tabs, src2d, dst2d)


def _stage_in(x, w, b):
    n = x.shape[0]
    blk = 400 if n % 400 == 0 else BLK

    def kern(x_ref, w_ref, b_ref, *outs):
        h = (jnp.dot(x_ref[...], w_ref[...],
                     preferred_element_type=jnp.float32) + b_ref[...])
        for c in range(NCHUNK):
            outs[c][...] = h[:, c * DC:(c + 1) * DC]

    return pl.pallas_call(
        kern,
        grid=(n // blk,),
        in_specs=[
            pl.BlockSpec((blk, D), lambda i: (i, 0)),
            pl.BlockSpec((D, D), lambda i: (0, 0)),
            pl.BlockSpec((1, D), lambda i: (0, 0)),
        ],
        out_specs=[pl.BlockSpec((blk, DC), lambda i: (i, 0))
                   for _ in range(NCHUNK)],
        out_shape=[jax.ShapeDtypeStruct((n, DC), jnp.float32)
                   for _ in range(NCHUNK)],
    )(x, w, b.reshape(1, D))


def _stage_merge(sums, cnt, w, b, relu, split_out):
    d_out = w.shape[1]

    def kern(s_ref, c_ref, w_ref, b_ref, *outs):
        hs = [s_ref[c, 0] + s_ref[c, 1] for c in range(NCHUNK)]
        h = jnp.concatenate(hs, axis=1)
        cnt_b = c_ref[0] + c_ref[1]
        h = h / jnp.maximum(cnt_b[:, 0:1], 1.0)
        if relu:
            h = jnp.where(h >= 0, h, 0.01 * h)
        o = (jnp.dot(h, w_ref[...], preferred_element_type=jnp.float32)
             + b_ref[...])
        if split_out:
            for c in range(NCHUNK):
                outs[c][...] = o[:, c * DC:(c + 1) * DC]
        else:
            outs[0][...] = o

    if split_out:
        out_specs = [pl.BlockSpec((BLK, DC), lambda i: (i, 0))
                     for _ in range(NCHUNK)]
        out_shape = [jax.ShapeDtypeStruct((NT, DC), jnp.float32)
                     for _ in range(NCHUNK)]
    else:
        out_specs = [pl.BlockSpec((BLK, d_out), lambda i: (i, 0))]
        out_shape = [jax.ShapeDtypeStruct((NT, d_out), jnp.float32)]

    return pl.pallas_call(
        kern,
        grid=(NT // BLK,),
        in_specs=[
            pl.BlockSpec((NCHUNK, NCORES, BLK, DC), lambda i: (0, 0, i, 0)),
            pl.BlockSpec((NCORES, BLK, DC), lambda i: (0, i, 0)),
            pl.BlockSpec((D, d_out), lambda i: (0, 0)),
            pl.BlockSpec((1, d_out), lambda i: (0, 0)),
        ],
        out_specs=out_specs,
        out_shape=out_shape,
    )(sums, cnt, w, b.reshape(1, d_out))


def _edges_pad(ei):
    npad = E_PAD - E
    src = jnp.concatenate([ei[0].astype(jnp.int32),
                           jnp.zeros((npad,), jnp.int32)])
    dst = jnp.concatenate([ei[1].astype(jnp.int32),
                           jnp.full((npad,), DUMP, jnp.int32)])
    return src, dst


def kernel(features, edge_index_u2t, edge_index_t2u, embed_user,
           W0_u2t, b0_u2t, W0_t2u, b0_t2u,
           W1_u2t, b1_u2t, W1_t2u, b1_t2u, Wc, bc):
    src_t2u, dst_t2u = _edges_pad(edge_index_t2u)
    src_u2t, dst_u2t = _edges_pad(edge_index_u2t)

    wh0 = _stage_in(features, W0_t2u, b0_t2u)
    sums_u, cnt_u = _seg_sums(wh0, src_t2u, dst_t2u)
    wh1 = _stage_merge(sums_u, cnt_u, W1_u2t, b1_u2t, relu=True,
                       split_out=True)
    sums_t, cnt_t = _seg_sums(wh1, src_u2t, dst_u2t)
    out = _stage_merge(sums_t, cnt_t, Wc, bc, relu=False, split_out=False)[0]
    return out[:N_NODES]


# speedup vs baseline: 1.8383x; 1.0610x over previous
import jax
import jax.numpy as jnp
from jax import lax
from jax.experimental import pallas as pl
from jax.experimental.pallas import tpu as pltpu
from jax.experimental.pallas import tpu_sc as plsc

N_NODES = 50000
D = 128
E = 300000
N_CLS = 8

NTILES = 16
NCORES = 2
NCHUNK = 8
DC = D // NCHUNK
KE = 256
KC = 128
NBATCH = 40
EPT = NBATCH * KE
E_PAD = EPT * NTILES * NCORES
ZR = 256
NZDMA = 13
TSTRIPE = 3136
NT = 50176
TB = NT + 8
DUMP = NT
STRIPE = NT // NTILES
BLK = 392


def _seg_sums(tabs, src2d, dst2d):
    def body(*refs):
        tabs_in = refs[:NCHUNK]
        (src_hbm, dst_hbm, sums_o, cnt_o,
         src_big, dst_big, rows, ones, zbuf, tab_sp, acc, sem) = refs[NCHUNK:]
        cid = lax.axis_index("c")
        sid = lax.axis_index("s")
        e0 = (cid * NTILES + sid) * EPT

        pltpu.sync_copy(src_hbm.at[pl.ds(e0, EPT)], src_big)
        pltpu.sync_copy(dst_hbm.at[pl.ds(e0, EPT)], dst_big)

        zf = jnp.zeros((16,), jnp.float32)
        of = jnp.ones((16,), jnp.float32)

        def init_row(r, _):
            zbuf[r, pl.ds(0, 16)] = zf
            return 0

        lax.fori_loop(0, ZR, init_row, 0)

        def init_ones(r, _):
            ones[r, pl.ds(0, 16)] = of
            return 0

        lax.fori_loop(0, KC, init_ones, 0)

        def zero_acc():
            z0 = sid * STRIPE
            for j in range(NZDMA):
                s = jnp.minimum(z0 + j * ZR, z0 + STRIPE - ZR)
                pltpu.sync_copy(zbuf, acc.at[pl.ds(s, ZR)])
            plsc.subcore_barrier()

        def writeout(dst_view):
            plsc.subcore_barrier()
            w0 = sid * STRIPE
            pltpu.sync_copy(acc.at[pl.ds(w0, STRIPE)],
                            dst_view.at[pl.ds(w0, STRIPE)])
            plsc.subcore_barrier()

        def zero_dump():
            pltpu.sync_copy(zbuf.at[pl.ds(0, 8)], acc.at[pl.ds(TB - 8, 8)])

        for c in range(NCHUNK):
            tab = tabs_in[c]
            t0 = jnp.minimum(sid * TSTRIPE, N_NODES - TSTRIPE)
            pltpu.sync_copy(tab.at[pl.ds(t0, TSTRIPE)],
                            tab_sp.at[pl.ds(t0, TSTRIPE)])
            zero_acc()
            zero_dump()
            plsc.subcore_barrier()

            def biter(b, _):
                pltpu.async_copy(
                    tab_sp.at[src_big.at[pl.ds(b * KE, KE)]], rows,
                    sem).wait()
                pltpu.sync_copy(rows, acc.at[dst_big.at[pl.ds(b * KE, KE)]],
                                add=True)
                return 0

            lax.fori_loop(0, NBATCH, biter, 0)
            writeout(sums_o.at[c, cid])

        zero_acc()
        zero_dump()

        def citer(b, _):
            pltpu.sync_copy(ones, acc.at[dst_big.at[pl.ds(b * KC, KC)]],
                            add=True)
            return 0

        lax.fori_loop(0, EPT // KC, citer, 0)
        writeout(cnt_o.at[cid])

    k = pl.kernel(
        body,
        out_type=(
            jax.ShapeDtypeStruct((NCHUNK, NCORES, NT, DC), jnp.float32),
            jax.ShapeDtypeStruct((NCORES, NT, DC), jnp.float32),
        ),
        mesh=plsc.VectorSubcoreMesh(core_axis_name="c", subcore_axis_name="s"),
        scratch_types=(
            pltpu.VMEM((EPT,), jnp.int32),
            pltpu.VMEM((EPT,), jnp.int32),
            pltpu.VMEM((KE, DC), jnp.float32),
            pltpu.VMEM((KC, DC), jnp.float32),
            pltpu.VMEM((ZR, DC), jnp.float32),
            pltpu.VMEM_SHARED((N_NODES, DC), jnp.float32),
            pltpu.VMEM_SHARED((TB, DC), jnp.float32),
            pltpu.SemaphoreType.DMA,
        ),
        compiler_params=pltpu.CompilerParams(use_tc_tiling_on_sc=False),
    )
    return k(*tabs, src2d, dst2d)


def _stage_in(x, w, b):
    n = x.shape[0]
    blk = 400 if n % 400 == 0 else BLK

    def kern(x_ref, w_ref, b_ref, *outs):
        h = (jnp.dot(x_ref[...], w_ref[...],
                     preferred_element_type=jnp.float32) + b_ref[...])
        for c in range(NCHUNK):
            outs[c][...] = h[:, c * DC:(c + 1) * DC]

    return pl.pallas_call(
        kern,
        grid=(n // blk,),
        in_specs=[
            pl.BlockSpec((blk, D), lambda i: (i, 0)),
            pl.BlockSpec((D, D), lambda i: (0, 0)),
            pl.BlockSpec((1, D), lambda i: (0, 0)),
        ],
        out_specs=[pl.BlockSpec((blk, DC), lambda i: (i, 0))
                   for _ in range(NCHUNK)],
        out_shape=[jax.ShapeDtypeStruct((n, DC), jnp.float32)
                   for _ in range(NCHUNK)],
    )(x, w, b.reshape(1, D))


def _stage_merge(sums, cnt, w, b, relu, split_out):
    d_out = w.shape[1]

    def kern(s_ref, c_ref, w_ref, b_ref, *outs):
        hs = [s_ref[c, 0] + s_ref[c, 1] for c in range(NCHUNK)]
        h = jnp.concatenate(hs, axis=1)
        cnt_b = c_ref[0] + c_ref[1]
        h = h / jnp.maximum(cnt_b[:, 0:1], 1.0)
        if relu:
            h = jnp.where(h >= 0, h, 0.01 * h)
        o = (jnp.dot(h, w_ref[...], preferred_element_type=jnp.float32)
             + b_ref[...])
        if split_out:
            for c in range(NCHUNK):
                outs[c][...] = o[:, c * DC:(c + 1) * DC]
        else:
            outs[0][...] = o

    if split_out:
        out_specs = [pl.BlockSpec((BLK, DC), lambda i: (i, 0))
                     for _ in range(NCHUNK)]
        out_shape = [jax.ShapeDtypeStruct((NT, DC), jnp.float32)
                     for _ in range(NCHUNK)]
    else:
        out_specs = [pl.BlockSpec((BLK, d_out), lambda i: (i, 0))]
        out_shape = [jax.ShapeDtypeStruct((NT, d_out), jnp.float32)]

    return pl.pallas_call(
        kern,
        grid=(NT // BLK,),
        in_specs=[
            pl.BlockSpec((NCHUNK, NCORES, BLK, DC), lambda i: (0, 0, i, 0)),
            pl.BlockSpec((NCORES, BLK, DC), lambda i: (0, i, 0)),
            pl.BlockSpec((D, d_out), lambda i: (0, 0)),
            pl.BlockSpec((1, d_out), lambda i: (0, 0)),
        ],
        out_specs=out_specs,
        out_shape=out_shape,
    )(sums, cnt, w, b.reshape(1, d_out))


def _edges_pad(ei):
    npad = E_PAD - E
    src = jnp.concatenate([ei[0].astype(jnp.int32),
                           jnp.zeros((npad,), jnp.int32)])
    dst = jnp.concatenate([ei[1].astype(jnp.int32),
                           jnp.full((npad,), DUMP, jnp.int32)])
    return src, dst


def kernel(features, edge_index_u2t, edge_index_t2u, embed_user,
           W0_u2t, b0_u2t, W0_t2u, b0_t2u,
           W1_u2t, b1_u2t, W1_t2u, b1_t2u, Wc, bc):
    src_t2u, dst_t2u = _edges_pad(edge_index_t2u)
    src_u2t, dst_u2t = _edges_pad(edge_index_u2t)

    wh0 = _stage_in(features, W0_t2u, b0_t2u)
    sums_u, cnt_u = _seg_sums(wh0, src_t2u, dst_t2u)
    wh1 = _stage_merge(sums_u, cnt_u, W1_u2t, b1_u2t, relu=True,
                       split_out=True)
    sums_t, cnt_t = _seg_sums(wh1, src_u2t, dst_u2t)
    out = _stage_merge(sums_t, cnt_t, Wc, bc, relu=False, split_out=False)[0]
    return out[:N_NODES]


# speedup vs baseline: 2.0924x; 1.1382x over previous
import jax
import jax.numpy as jnp
from jax import lax
from jax.experimental import pallas as pl
from jax.experimental.pallas import tpu as pltpu
from jax.experimental.pallas import tpu_sc as plsc

N_NODES = 50000
D = 128
E = 300000
N_CLS = 8

NTILES = 16
NCORES = 2
NCHUNK = 8
DC = D // NCHUNK
KE = 256
KC = 128
NBATCH = 38
EPT = NBATCH * KE
E_PAD = EPT * NTILES * NCORES
TSTRIPE = 3136
NT = 50176
TB = NT + 8
DUMP = NT
STRIPE = NT // NTILES
BLK = 392


def _seg_sums(tabs, src2d, dst2d):
    def body(*refs):
        tabs_in = refs[:NCHUNK]
        (src_hbm, dst_hbm, zeros_hbm, ones_hbm, sums_o, cnt_o,
         src_big, dst_big, rows0, rows1, ones, tab_sp, acc,
         sem0, sem1) = refs[NCHUNK:]
        rows = (rows0, rows1)
        sems = (sem0, sem1)
        cid = lax.axis_index("c")
        sid = lax.axis_index("s")
        e0 = (cid * NTILES + sid) * EPT

        pltpu.sync_copy(src_hbm.at[pl.ds(e0, EPT + KE)], src_big)
        pltpu.sync_copy(dst_hbm.at[pl.ds(e0, EPT)], dst_big)
        pltpu.sync_copy(ones_hbm, ones)

        def zero_acc():
            z0 = sid * STRIPE
            pltpu.sync_copy(zeros_hbm, acc.at[pl.ds(z0, STRIPE)])
            pltpu.sync_copy(zeros_hbm.at[pl.ds(0, 8)],
                            acc.at[pl.ds(TB - 8, 8)])
            plsc.subcore_barrier()

        def writeout(dst_view):
            plsc.subcore_barrier()
            w0 = sid * STRIPE
            pltpu.sync_copy(acc.at[pl.ds(w0, STRIPE)],
                            dst_view.at[pl.ds(w0, STRIPE)])
            plsc.subcore_barrier()

        for c in range(NCHUNK):
            tab = tabs_in[c]
            t0 = jnp.minimum(sid * TSTRIPE, N_NODES - TSTRIPE)
            pltpu.sync_copy(tab.at[pl.ds(t0, TSTRIPE)],
                            tab_sp.at[pl.ds(t0, TSTRIPE)])
            zero_acc()

            def gdesc(b, p, make=False):
                f = pltpu.make_async_copy if make else pltpu.async_copy
                return f(tab_sp.at[src_big.at[pl.ds(b * KE, KE)]], rows[p],
                         sems[p])

            def scat(b, p):
                pltpu.sync_copy(rows[p],
                                acc.at[dst_big.at[pl.ds(b * KE, KE)]],
                                add=True)

            gdesc(0, 0)

            def biter2(i, _):
                b0 = i * 2
                gdesc(b0, 0, make=True).wait()
                gdesc(b0 + 1, 1)
                scat(b0, 0)
                gdesc(b0 + 1, 1, make=True).wait()
                gdesc(b0 + 2, 0)
                scat(b0 + 1, 1)
                return 0

            lax.fori_loop(0, NBATCH // 2, biter2, 0)
            gdesc(NBATCH, 0, make=True).wait()
            writeout(sums_o.at[c, cid])

        zero_acc()

        def citer(b, _):
            pltpu.sync_copy(ones, acc.at[dst_big.at[pl.ds(b * KC, KC)]],
                            add=True)
            return 0

        lax.fori_loop(0, EPT // KC, citer, 0)
        writeout(cnt_o.at[cid])

    k = pl.kernel(
        body,
        out_type=(
            jax.ShapeDtypeStruct((NCHUNK, NCORES, NT, DC), jnp.float32),
            jax.ShapeDtypeStruct((NCORES, NT, DC), jnp.float32),
        ),
        mesh=plsc.VectorSubcoreMesh(core_axis_name="c", subcore_axis_name="s"),
        scratch_types=(
            pltpu.VMEM((EPT + KE,), jnp.int32),
            pltpu.VMEM((EPT,), jnp.int32),
            pltpu.VMEM((KE, DC), jnp.float32),
            pltpu.VMEM((KE, DC), jnp.float32),
            pltpu.VMEM((KC, DC), jnp.float32),
            pltpu.VMEM_SHARED((N_NODES, DC), jnp.float32),
            pltpu.VMEM_SHARED((TB, DC), jnp.float32),
            pltpu.SemaphoreType.DMA,
            pltpu.SemaphoreType.DMA,
        ),
        compiler_params=pltpu.CompilerParams(use_tc_tiling_on_sc=False),
    )
    return k(*tabs, src2d, dst2d, _zeros_const(), _ones_const())


def _stage_in(x, w, b):
    n = x.shape[0]
    blk = 400 if n % 400 == 0 else BLK

    def kern(x_ref, w_ref, b_ref, *outs):
        h = (jnp.dot(x_ref[...], w_ref[...],
                     preferred_element_type=jnp.float32) + b_ref[...])
        for c in range(NCHUNK):
            outs[c][...] = h[:, c * DC:(c + 1) * DC]

    return pl.pallas_call(
        kern,
        grid=(n // blk,),
        in_specs=[
            pl.BlockSpec((blk, D), lambda i: (i, 0)),
            pl.BlockSpec((D, D), lambda i: (0, 0)),
            pl.BlockSpec((1, D), lambda i: (0, 0)),
        ],
        out_specs=[pl.BlockSpec((blk, DC), lambda i: (i, 0))
                   for _ in range(NCHUNK)],
        out_shape=[jax.ShapeDtypeStruct((n, DC), jnp.float32)
                   for _ in range(NCHUNK)],
    )(x, w, b.reshape(1, D))


def _stage_merge(sums, cnt, w, b, relu, split_out):
    d_out = w.shape[1]

    def kern(s_ref, c_ref, w_ref, b_ref, *outs):
        hs = [s_ref[c, 0] + s_ref[c, 1] for c in range(NCHUNK)]
        h = jnp.concatenate(hs, axis=1)
        cnt_b = c_ref[0] + c_ref[1]
        h = h / jnp.maximum(cnt_b[:, 0:1], 1.0)
        if relu:
            h = jnp.where(h >= 0, h, 0.01 * h)
        o = (jnp.dot(h, w_ref[...], preferred_element_type=jnp.float32)
             + b_ref[...])
        if split_out:
            for c in range(NCHUNK):
                outs[c][...] = o[:, c * DC:(c + 1) * DC]
        else:
            outs[0][...] = o

    if split_out:
        out_specs = [pl.BlockSpec((BLK, DC), lambda i: (i, 0))
                     for _ in range(NCHUNK)]
        out_shape = [jax.ShapeDtypeStruct((NT, DC), jnp.float32)
                     for _ in range(NCHUNK)]
    else:
        out_specs = [pl.BlockSpec((BLK, d_out), lambda i: (i, 0))]
        out_shape = [jax.ShapeDtypeStruct((NT, d_out), jnp.float32)]

    return pl.pallas_call(
        kern,
        grid=(NT // BLK,),
        in_specs=[
            pl.BlockSpec((NCHUNK, NCORES, BLK, DC), lambda i: (0, 0, i, 0)),
            pl.BlockSpec((NCORES, BLK, DC), lambda i: (0, i, 0)),
            pl.BlockSpec((D, d_out), lambda i: (0, 0)),
            pl.BlockSpec((1, d_out), lambda i: (0, 0)),
        ],
        out_specs=out_specs,
        out_shape=out_shape,
    )(sums, cnt, w, b.reshape(1, d_out))


def _edges_pad(ei):
    src = jnp.concatenate([ei[0].astype(jnp.int32),
                           jnp.zeros((E_PAD + KE - E,), jnp.int32)])
    dst = jnp.concatenate([ei[1].astype(jnp.int32),
                           jnp.full((E_PAD - E,), DUMP, jnp.int32)])
    return src, dst


def _zeros_const():
    return jnp.zeros((STRIPE, DC), jnp.float32)


def _ones_const():
    return jnp.ones((KC, DC), jnp.float32)


def kernel(features, edge_index_u2t, edge_index_t2u, embed_user,
           W0_u2t, b0_u2t, W0_t2u, b0_t2u,
           W1_u2t, b1_u2t, W1_t2u, b1_t2u, Wc, bc):
    src_t2u, dst_t2u = _edges_pad(edge_index_t2u)
    src_u2t, dst_u2t = _edges_pad(edge_index_u2t)

    wh0 = _stage_in(features, W0_t2u, b0_t2u)
    sums_u, cnt_u = _seg_sums(wh0, src_t2u, dst_t2u)
    wh1 = _stage_merge(sums_u, cnt_u, W1_u2t, b1_u2t, relu=True,
                       split_out=True)
    sums_t, cnt_t = _seg_sums(wh1, src_u2t, dst_u2t)
    out = _stage_merge(sums_t, cnt_t, Wc, bc, relu=False, split_out=False)[0]
    return out[:N_NODES]
